# baseline (speedup 1.0000x reference)
import functools

import jax
import jax.numpy as jnp
from jax import lax
from jax.experimental import pallas as pl
from jax.experimental.pallas import tpu as pltpu
from jax.experimental.pallas import tpu_sc as plsc

D = 64
L = 16
HIST = 200
SUPER = 16
NC, NS = 2, 16
NW = NC * NS
NGRP = (HIST + L - 1) // L
SIDX = SUPER * HIST


def _body(idx_hbm, table_hbm, out_hbm, idx_v, table_v, rows_v, isem, osem,
          *, batch):
    bpw = batch // NW
    n_super = bpw // SUPER
    wid = lax.axis_index("s") * NC + lax.axis_index("c")
    i0 = wid * bpw * HIST

    pltpu.sync_copy(table_hbm, table_v)

    iota = lax.iota(jnp.int32, L)
    tail_mask = iota < (HIST - (NGRP - 1) * L)

    def idx_cp(s, ib):
        return pltpu.make_async_copy(
            idx_hbm.at[pl.ds(i0 + s * SIDX, SIDX)],
            idx_v.at[pl.ds(ib * SIDX, SIDX)],
            isem,
        )

    def write_cp(c, b):
        return pltpu.make_async_copy(rows_v.at[c % 2], out_hbm.at[b], osem)

    def compute_chunk(c, base):
        cb = c % 2
        for g in range(NGRP):
            full = g < NGRP - 1
            n = L if full else HIST - g * L
            mask = None if full else tail_mask
            ivec = idx_v[pl.ds(base + g * L, L)]
            fl = ivec * D
            rvec = iota + g * L
            for l in range(D):
                vals = plsc.load_gather(table_v, [fl + l], mask=mask)
                plsc.store_scatter(
                    rows_v.at[cb], [rvec, iota * 0 + l], vals, mask=mask
                )

    def outer(s, _):
        ib = s % 2

        idx_cp(s, ib).wait()

        @pl.when(s + 1 < n_super)
        def _():
            idx_cp(s + 1, (s + 1) % 2).start()

        def inner(j, _):
            c = s * SUPER + j

            @pl.when(c >= 2)
            def _():
                write_cp(c, 0).wait()

            compute_chunk(c, ib * SIDX + j * HIST)
            write_cp(c, wid * bpw + c).start()
            return 0

        lax.fori_loop(0, SUPER, inner, 0)
        return 0

    idx_cp(0, 0).start()
    lax.fori_loop(0, n_super, outer, 0)
    write_cp(bpw - 2, 0).wait()
    write_cp(bpw - 1, 0).wait()


@functools.partial(jax.jit, static_argnames=("batch", "hist"))
def _gather(idx, table, *, batch, hist):
    body = functools.partial(_body, batch=batch)
    return pl.kernel(
        body,
        out_type=jax.ShapeDtypeStruct((batch, hist, D), jnp.float32),
        mesh=plsc.VectorSubcoreMesh(core_axis_name="c", subcore_axis_name="s"),
        scratch_types=[
            pltpu.VMEM((2 * SIDX + L,), jnp.int32),
            pltpu.VMEM(((1000 + 1) * D,), jnp.float32),
            pltpu.VMEM((2, HIST, D), jnp.float32),
            pltpu.SemaphoreType.DMA,
            pltpu.SemaphoreType.DMA,
        ],
        compiler_params=pltpu.CompilerParams(
            use_tc_tiling_on_sc=True, needs_layout_passes=False
        ),
    )(idx, table)


def kernel(x, W):
    b, h = x.shape
    idx = x.reshape(b * h).astype(jnp.int32)
    table = W.reshape(-1)
    return _gather(idx, table, batch=b, hist=h)


# speedup vs baseline: 2.0595x; 2.0595x over previous
import functools

import jax
import jax.numpy as jnp
from jax import lax
from jax.experimental import pallas as pl
from jax.experimental.pallas import tpu as pltpu
from jax.experimental.pallas import tpu_sc as plsc

D = 64
L = 16
HIST = 200
SUPER = 16
NC, NS = 2, 16
NW = NC * NS
NGRP = (HIST + L - 1) // L
SIDX = SUPER * HIST


def _body(idx_hbm, table_hbm, out_hbm, idx_v, table_v, rows_v, isem, osem,
          *, batch):
    bpw = batch // NW
    n_super = bpw // SUPER
    wid = lax.axis_index("s") * NC + lax.axis_index("c")
    i0 = wid * bpw * HIST

    pltpu.sync_copy(table_hbm, table_v)

    iota = lax.iota(jnp.int32, L)
    tail_mask = iota < (HIST - (NGRP - 1) * L)

    def idx_cp(s, ib):
        return pltpu.make_async_copy(
            idx_hbm.at[pl.ds(i0 + s * SIDX, SIDX)],
            idx_v.at[pl.ds(ib * SIDX, SIDX)],
            isem,
        )

    def write_cp(c, b):
        return pltpu.make_async_copy(rows_v.at[c % 2], out_hbm.at[b], osem)

    zero = iota * 0

    def compute_chunk(c, base):
        cb = c % 2
        for g in range(NGRP):
            full = g < NGRP - 1
            mask = None if full else tail_mask
            ivec = idx_v[pl.ds(base + g * L, L)]
            fl = ivec * D
            rvec = iota + g * L

            @plsc.parallel_loop(0, D, unroll=8)
            def _(l):
                vals = plsc.load_gather(table_v, [fl + l], mask=mask)
                plsc.store_scatter(
                    rows_v.at[cb], [rvec, zero + l], vals, mask=mask
                )

    def outer(s, _):
        ib = s % 2

        idx_cp(s, ib).wait()

        @pl.when(s + 1 < n_super)
        def _():
            idx_cp(s + 1, (s + 1) % 2).start()

        def inner(j, _):
            c = s * SUPER + j

            @pl.when(c >= 2)
            def _():
                write_cp(c, 0).wait()

            compute_chunk(c, ib * SIDX + j * HIST)
            write_cp(c, wid * bpw + c).start()
            return 0

        lax.fori_loop(0, SUPER, inner, 0)
        return 0

    idx_cp(0, 0).start()
    lax.fori_loop(0, n_super, outer, 0)
    write_cp(bpw - 2, 0).wait()
    write_cp(bpw - 1, 0).wait()


@functools.partial(jax.jit, static_argnames=("batch", "hist"))
def _gather(idx, table, *, batch, hist):
    body = functools.partial(_body, batch=batch)
    return pl.kernel(
        body,
        out_type=jax.ShapeDtypeStruct((batch, hist, D), jnp.float32),
        mesh=plsc.VectorSubcoreMesh(core_axis_name="c", subcore_axis_name="s"),
        scratch_types=[
            pltpu.VMEM((2 * SIDX + L,), jnp.int32),
            pltpu.VMEM(((1000 + 1) * D,), jnp.float32),
            pltpu.VMEM((2, HIST, D), jnp.float32),
            pltpu.SemaphoreType.DMA,
            pltpu.SemaphoreType.DMA,
        ],
        compiler_params=pltpu.CompilerParams(
            use_tc_tiling_on_sc=True, needs_layout_passes=False
        ),
    )(idx, table)


def kernel(x, W):
    b, h = x.shape
    idx = x.reshape(b * h).astype(jnp.int32)
    table = W.reshape(-1)
    return _gather(idx, table, batch=b, hist=h)


# speedup vs baseline: 19.2108x; 9.3279x over previous
import functools

import jax
import jax.numpy as jnp
from jax import lax
from jax.experimental import pallas as pl
from jax.experimental.pallas import tpu as pltpu
from jax.experimental.pallas import tpu_sc as plsc

D = 64
L = 16
BSL = 512
HALF = 256
NC, NS = 2, 16
NW = NC * NS


def _body(idx_hbm, wt_hbm, out_hbm, idx_v, wt_v, buf_v, isem, osem,
          *, batch, hist):
    wid = lax.axis_index("s") * NC + lax.axis_index("c")
    b0 = wid * BSL

    pltpu.sync_copy(wt_hbm, wt_v)

    iota = lax.iota(jnp.int32, L)
    zero = iota * 0

    def idx_cp(h, ib):
        return pltpu.make_async_copy(
            idx_hbm.at[pl.ds(h * batch + b0, BSL)],
            idx_v.at[pl.ds(ib * BSL, BSL)],
            isem,
        )

    def write_cp(c, h, t):
        return pltpu.make_async_copy(
            buf_v.at[c % 2],
            out_hbm.at[h, :, pl.ds(b0 + t * HALF, HALF)],
            osem,
        )

    def compute_half(c, ioff):
        cb = c % 2
        ivecs = [idx_v[pl.ds(ioff + g * L, L)] for g in range(HALF // L)]

        @plsc.parallel_loop(0, D, unroll=2)
        def _(d):
            dvec = zero + d
            for g in range(HALF // L):
                vals = plsc.load_gather(wt_v, [dvec, ivecs[g]])
                buf_v[cb, d, pl.ds(g * L, L)] = vals

    def hloop(h, _):
        ib = h % 2
        idx_cp(h, ib).wait()

        @pl.when(h + 1 < hist)
        def _():
            idx_cp(h + 1, (h + 1) % 2).start()

        for t in range(2):
            c = h * 2 + t

            @pl.when(c >= 2)
            def _():
                write_cp(c, 0, 0).wait()

            compute_half(c, ib * BSL + t * HALF)
            write_cp(c, h, t).start()
        return 0

    idx_cp(0, 0).start()
    lax.fori_loop(0, hist, hloop, 0)
    write_cp(0, 0, 0).wait()
    write_cp(1, 0, 0).wait()


@functools.partial(jax.jit, static_argnames=("batch", "hist"))
def _gather(idx, wt, *, batch, hist):
    body = functools.partial(_body, batch=batch, hist=hist)
    return pl.kernel(
        body,
        out_type=jax.ShapeDtypeStruct((hist, D, batch), jnp.float32),
        mesh=plsc.VectorSubcoreMesh(core_axis_name="c", subcore_axis_name="s"),
        scratch_types=[
            pltpu.VMEM((2 * BSL,), jnp.int32),
            pltpu.VMEM((D, 1000 + 1), jnp.float32),
            pltpu.VMEM((2, D, HALF), jnp.float32),
            pltpu.SemaphoreType.DMA,
            pltpu.SemaphoreType.DMA,
        ],
        compiler_params=pltpu.CompilerParams(
            use_tc_tiling_on_sc=True, needs_layout_passes=False
        ),
    )(idx, wt)


def kernel(x, W):
    b, h = x.shape
    idx = x.T.reshape(h * b).astype(jnp.int32)
    wt = W.T
    out = _gather(idx, wt, batch=b, hist=h)
    return jnp.transpose(out, (2, 0, 1))
